# Initial kernel scaffold; baseline (speedup 1.0000x reference)
import jax
import jax.numpy as jnp
from jax.experimental import pallas as pl


def kernel(feats, cluster_ids, emb, W1, b1, W2, b2, W3, b3):
    raise NotImplementedError("write your pallas kernel here")


# speedup vs baseline: 2.3890x; 2.3890x over previous
import functools

import jax
import jax.numpy as jnp
from jax import lax
from jax.experimental import pallas as pl
from jax.experimental.pallas import tpu as pltpu
from jax.experimental.pallas import tpu_sc as plsc

_EPS = 1e-10


def _mlp_body(emb_ref, w1t_ref, b1_ref, w2t_ref, b2_ref, w3t_ref, b3_ref,
              u0_ref, u1_ref, out_ref):
    h = jnp.dot(emb_ref[...], w1t_ref[...],
                preferred_element_type=jnp.float32,
                precision=lax.Precision.HIGHEST)
    h = jnp.maximum(h + b1_ref[...], 0.0)
    h = jnp.dot(h, w2t_ref[...],
                preferred_element_type=jnp.float32,
                precision=lax.Precision.HIGHEST)
    h = jnp.maximum(h + b2_ref[...], 0.0)
    cf = jnp.dot(h, w3t_ref[...],
                 preferred_element_type=jnp.float32,
                 precision=lax.Precision.HIGHEST)
    cf = cf + b3_ref[0, 0]
    noise = -jnp.log(jnp.log(u1_ref[...]) / jnp.log(u0_ref[...]) + _EPS)
    logits = cf + noise
    out_ref[...] = 1.0 / (1.0 + jnp.exp(-logits))


def _cluster_probs(emb, W1, b1, W2, b2, W3, b3, u0, u1):
    n, d = emb.shape
    return pl.pallas_call(
        _mlp_body,
        out_shape=jax.ShapeDtypeStruct((n, 1), jnp.float32),
    )(emb, W1.T, b1.reshape(1, d), W2.T, b2.reshape(1, d),
      W3.T, b3.reshape(1, 1), u0, u1)


@functools.cache
def _make_gather(batch: int, table_pad: int):
    info = plsc.get_sparse_core_info()
    nc, ns = info.num_cores, info.num_subcores
    nw = nc * ns
    bpw = batch // nw
    mesh = plsc.VectorSubcoreMesh(core_axis_name="c", subcore_axis_name="s")

    chunk = 128
    nchunks = bpw // chunk

    @functools.partial(
        pl.kernel,
        mesh=mesh,
        out_type=jax.ShapeDtypeStruct((batch,), jnp.float32),
        scratch_types=[
            pltpu.VMEM((bpw,), jnp.int32),
            pltpu.VMEM((bpw,), jnp.float32),
            pltpu.SemaphoreType.DMA,
        ],
    )
    def gather_k(table_hbm, idx_hbm, out_hbm, idx_v, out_v, sem):
        wid = lax.axis_index("s") * nc + lax.axis_index("c")
        base = wid * bpw
        pltpu.sync_copy(idx_hbm.at[pl.ds(base, bpw)], idx_v)
        descs = []
        for j in range(nchunks):
            sl = pl.ds(j * chunk, chunk)
            descs.append(
                pltpu.async_copy(table_hbm.at[idx_v.at[sl]], out_v.at[sl], sem))
        for d in descs:
            d.wait()
        pltpu.sync_copy(out_v, out_hbm.at[pl.ds(base, bpw)])

    return gather_k


def kernel(feats, cluster_ids, emb, W1, b1, W2, b2, W3, b3):
    n_clusters = emb.shape[0]
    batch = cluster_ids.shape[0]
    u = jax.random.uniform(jax.random.key(42), (2, n_clusters, 1),
                           dtype=jnp.float32, minval=_EPS, maxval=1.0 - _EPS)
    probs = _cluster_probs(emb, W1, b1, W2, b2, W3, b3, u[0], u[1])
    table_pad = ((n_clusters + 1023) // 1024) * 1024
    table = jnp.pad(probs[:, 0], (0, table_pad - n_clusters))
    out = _make_gather(batch, table_pad)(table, cluster_ids[:, 0])
    return out[:, None]


# speedup vs baseline: 2.9802x; 1.2475x over previous
import functools

import jax
import jax.numpy as jnp
import numpy as np
from jax import lax
from jax.experimental import pallas as pl
from jax.experimental.pallas import tpu as pltpu
from jax.experimental.pallas import tpu_sc as plsc

_EPS = 1e-10


def _gumbel_noise(n: int):
    u = jax.random.uniform(jax.random.key(42), (2, n, 1), dtype=jnp.float32,
                           minval=_EPS, maxval=1.0 - _EPS)
    return -jnp.log(jnp.log(u[1]) / jnp.log(u[0]) + _EPS)


_DN = (((1,), (1,)), ((), ()))


def _mlp_body(emb_ref, w1_ref, b1_ref, w2_ref, b2_ref, w3_ref, b3_ref,
              noise_ref, out_ref):
    h = lax.dot_general(emb_ref[...], w1_ref[...], _DN,
                        preferred_element_type=jnp.float32)
    h = jnp.maximum(h + b1_ref[...], 0.0)
    h = lax.dot_general(h, w2_ref[...], _DN,
                        preferred_element_type=jnp.float32)
    h = jnp.maximum(h + b2_ref[...], 0.0)
    cf = jnp.sum(h * w3_ref[...], axis=1, keepdims=True) + b3_ref[0, 0]
    logits = cf + noise_ref[...]
    n = logits.shape[0]
    out_ref[pl.ds(0, n), :] = 1.0 / (1.0 + jnp.exp(-logits))
    out_ref[pl.ds(n, out_ref.shape[0] - n), :] = jnp.zeros(
        (out_ref.shape[0] - n, 1), jnp.float32)


def _cluster_probs(emb, W1, b1, W2, b2, W3, b3, noise, n_pad):
    n, d = emb.shape
    return pl.pallas_call(
        _mlp_body,
        out_shape=jax.ShapeDtypeStruct((n_pad, 1), jnp.float32),
    )(emb, W1, b1.reshape(1, d), W2, b2.reshape(1, d),
      W3, b3.reshape(1, 1), noise)


@functools.cache
def _make_gather(batch: int, table_pad: int):
    info = plsc.get_sparse_core_info()
    nc, ns = info.num_cores, info.num_subcores
    nw = nc * ns
    bpw = batch // nw
    mesh = plsc.VectorSubcoreMesh(core_axis_name="c", subcore_axis_name="s")

    chunk = 128
    nchunks = bpw // chunk

    @functools.partial(
        pl.kernel,
        mesh=mesh,
        out_type=jax.ShapeDtypeStruct((batch,), jnp.float32),
        scratch_types=[
            pltpu.VMEM((bpw,), jnp.int32),
            pltpu.VMEM((bpw,), jnp.float32),
            pltpu.SemaphoreType.DMA,
        ],
    )
    def gather_k(table_hbm, idx_hbm, out_hbm, idx_v, out_v, sem):
        wid = lax.axis_index("s") * nc + lax.axis_index("c")
        base = wid * bpw
        pltpu.sync_copy(idx_hbm.at[pl.ds(base, bpw)], idx_v)
        descs = []
        for j in range(nchunks):
            sl = pl.ds(j * chunk, chunk)
            descs.append(
                pltpu.async_copy(table_hbm.at[idx_v.at[sl]], out_v.at[sl], sem))
        for d in descs:
            d.wait()
        pltpu.sync_copy(out_v, out_hbm.at[pl.ds(base, bpw)])

    return gather_k


def kernel(feats, cluster_ids, emb, W1, b1, W2, b2, W3, b3):
    n_clusters = emb.shape[0]
    batch = cluster_ids.shape[0]
    table_pad = ((n_clusters + 1023) // 1024) * 1024
    noise = _gumbel_noise(n_clusters)
    probs = _cluster_probs(emb, W1, b1, W2, b2, W3, b3, noise, table_pad)
    out = _make_gather(batch, table_pad)(probs[:, 0], cluster_ids[:, 0])
    return out[:, None]


# speedup vs baseline: 4.5066x; 1.5122x over previous
import functools

import jax
import jax.numpy as jnp
from jax import lax
from jax.experimental import pallas as pl
from jax.experimental.pallas import tpu as pltpu
from jax.experimental.pallas import tpu_sc as plsc

_EPS = 1e-10


_DN = (((1,), (1,)), ((), ()))


def _mlp_body(emb_ref, w1_ref, b1_ref, w2_ref, b2_ref, w3_ref, b3_ref,
              u_ref, out_ref):
    h = lax.dot_general(emb_ref[...], w1_ref[...], _DN,
                        preferred_element_type=jnp.float32)
    h = jnp.maximum(h + b1_ref[...], 0.0)
    h = lax.dot_general(h, w2_ref[...], _DN,
                        preferred_element_type=jnp.float32)
    h = jnp.maximum(h + b2_ref[...], 0.0)
    cf = jnp.sum(h * w3_ref[...], axis=1) + b3_ref[0, 0]
    noise = -jnp.log(jnp.log(u_ref[1, :]) / jnp.log(u_ref[0, :]) + _EPS)
    logits = cf + noise
    n = logits.shape[0]
    out_ref[pl.ds(0, n)] = 1.0 / (1.0 + jnp.exp(-logits))
    out_ref[pl.ds(n, out_ref.shape[0] - n)] = jnp.zeros(
        (out_ref.shape[0] - n,), jnp.float32)


def _cluster_probs(emb, W1, b1, W2, b2, W3, b3, u2, n_pad):
    n, d = emb.shape
    return pl.pallas_call(
        _mlp_body,
        out_shape=jax.ShapeDtypeStruct((n_pad,), jnp.float32),
    )(emb, W1, b1.reshape(1, d), W2, b2.reshape(1, d),
      W3, b3.reshape(1, 1), u2)


@functools.cache
def _make_gather(batch: int, table_pad: int):
    info = plsc.get_sparse_core_info()
    nc, ns = info.num_cores, info.num_subcores
    nw = nc * ns
    bpw = batch // nw
    mesh = plsc.VectorSubcoreMesh(core_axis_name="c", subcore_axis_name="s")

    chunk = 128
    nchunks = bpw // chunk

    @functools.partial(
        pl.kernel,
        mesh=mesh,
        out_type=jax.ShapeDtypeStruct((batch,), jnp.float32),
        scratch_types=[
            pltpu.VMEM_SHARED((table_pad,), jnp.float32),
            pltpu.VMEM((bpw,), jnp.int32),
            pltpu.VMEM((bpw,), jnp.float32),
            pltpu.SemaphoreType.DMA,
            pltpu.SemaphoreType.DMA,
        ],
    )
    def gather_k(table_hbm, idx_hbm, out_hbm, table_sh, idx_v, out_v, sem,
                 idx_sem):
        wid = lax.axis_index("s") * nc + lax.axis_index("c")
        base = wid * bpw
        idx_cp = pltpu.async_copy(idx_hbm.at[pl.ds(base, bpw)], idx_v,
                                  idx_sem)

        @pl.when(lax.axis_index("s") == 0)
        def _():
            pltpu.sync_copy(table_hbm, table_sh)

        plsc.subcore_barrier()
        idx_cp.wait()
        descs = []
        for j in range(nchunks):
            sl = pl.ds(j * chunk, chunk)
            descs.append(
                pltpu.async_copy(table_sh.at[idx_v.at[sl]], out_v.at[sl],
                                 sem))
        for d in descs:
            d.wait()
        pltpu.sync_copy(out_v, out_hbm.at[pl.ds(base, bpw)])

    return gather_k


def kernel(feats, cluster_ids, emb, W1, b1, W2, b2, W3, b3):
    n_clusters = emb.shape[0]
    batch = cluster_ids.shape[0]
    table_pad = ((n_clusters + 1023) // 1024) * 1024
    u = jax.random.uniform(jax.random.key(42), (2, n_clusters, 1),
                           dtype=jnp.float32, minval=_EPS, maxval=1.0 - _EPS)
    table = _cluster_probs(emb, W1, b1, W2, b2, W3, b3, u[:, :, 0],
                           table_pad)
    out = _make_gather(batch, table_pad)(table, cluster_ids[:, 0])
    return out[:, None]
